# Initial kernel scaffold; baseline (speedup 1.0000x reference)
import jax
import jax.numpy as jnp
from jax.experimental import pallas as pl


def kernel(x, edge_index, W1, b1, W2, b2, Wfc, bfc):
    raise NotImplementedError("write your pallas kernel here")


# speedup vs baseline: 9.2740x; 9.2740x over previous
import functools

import jax
import jax.numpy as jnp
from jax import lax
from jax.experimental import pallas as pl
from jax.experimental.pallas import tpu as pltpu
from jax.experimental.pallas import tpu_sc as plsc

NC = 2
NT = 16
CH = 125
BN = 1000


def _segsum_sc(D, table, srcidx, dstidx, zeros):
    nch = srcidx.shape[2]
    NP = zeros.shape[0]
    rpt = NP // NT
    mesh = plsc.VectorSubcoreMesh(core_axis_name="c", subcore_axis_name="s")

    @functools.partial(
        pl.kernel,
        mesh=mesh,
        out_type=jax.ShapeDtypeStruct((NC, NP, D), jnp.float32),
        scratch_types=[
            pltpu.VMEM((nch, CH), jnp.int32),
            pltpu.VMEM((nch, CH), jnp.int32),
            pltpu.VMEM((CH, D), jnp.float32),
            pltpu.VMEM_SHARED((NP, D), jnp.float32),
            pltpu.SemaphoreType.DMA,
        ],
    )
    def seg(table_hbm, src_hbm, dst_hbm, zeros_hbm, out_hbm,
            sidx, didx, gbuf, acc, sem):
        c = lax.axis_index("c")
        s = lax.axis_index("s")
        pltpu.sync_copy(src_hbm.at[c, s], sidx)
        pltpu.sync_copy(dst_hbm.at[c, s], didx)

        @pl.when(s == 0)
        def _():
            pltpu.sync_copy(zeros_hbm, acc)

        plsc.subcore_barrier()

        def body(j, carry):
            pltpu.async_copy(table_hbm.at[sidx.at[j]], gbuf, sem).wait()
            pltpu.sync_copy(gbuf, acc.at[didx.at[j]], add=True)
            return carry

        lax.fori_loop(0, nch, body, 0)
        plsc.subcore_barrier()
        pltpu.sync_copy(acc.at[pl.ds(s * rpt, rpt)],
                        out_hbm.at[c, pl.ds(s * rpt, rpt)])

    return seg(table, srcidx, dstidx, zeros)


def _deg_sc(dstidx, zerosD, onesD):
    nch = dstidx.shape[2]
    NP, D = zerosD.shape
    rpt = NP // NT
    mesh = plsc.VectorSubcoreMesh(core_axis_name="c", subcore_axis_name="s")

    @functools.partial(
        pl.kernel,
        mesh=mesh,
        out_type=jax.ShapeDtypeStruct((NC, NP, D), jnp.float32),
        scratch_types=[
            pltpu.VMEM((nch, CH), jnp.int32),
            pltpu.VMEM((CH, D), jnp.float32),
            pltpu.VMEM_SHARED((NP, D), jnp.float32),
        ],
    )
    def deg(dst_hbm, zeros_hbm, ones_hbm, out_hbm, didx, onesb, acc):
        c = lax.axis_index("c")
        s = lax.axis_index("s")
        pltpu.sync_copy(dst_hbm.at[c, s], didx)
        pltpu.sync_copy(ones_hbm, onesb)

        @pl.when(s == 0)
        def _():
            pltpu.sync_copy(zeros_hbm, acc)

        plsc.subcore_barrier()

        def body(j, carry):
            pltpu.sync_copy(onesb, acc.at[didx.at[j]], add=True)
            return carry

        lax.fori_loop(0, nch, body, 0)
        plsc.subcore_barrier()
        pltpu.sync_copy(acc.at[pl.ds(s * rpt, rpt)],
                        out_hbm.at[c, pl.ds(s * rpt, rpt)])

    return deg(dstidx, zerosD, onesD)


def _norm_col(d0, d1):
    deg = d0[0, :, 0:1] + d1[0, :, 0:1]
    return lax.rsqrt(jnp.maximum(deg, 1.0))


def _prep_body(x_ref, d0, d1, y_ref):
    y_ref[...] = x_ref[...] * _norm_col(d0, d1)


def _zscale_body(sa, sb, d0, d1, z_ref):
    deg = d0[0, :, 0:1] + d1[0, :, 0:1]
    dinv = 1.0 / jnp.maximum(deg, 1.0)
    z_ref[...] = (sa[0] + sb[0]) * dinv


def _dot(a, b):
    return jnp.dot(a, b, preferred_element_type=jnp.float32)


def _combine1_body(x_ref, s1a, s1b, s2a, s2b, d0, d1,
                   w0, w1, w2, b1, h_ref, y_ref):
    nrm = _norm_col(d0, d1)
    x = x_ref[...]
    p1 = (s1a[0] + s1b[0]) * nrm
    p2 = (s2a[0] + s2b[0]) * nrm
    h = (_dot(x, w0[0]) - _dot(p1, w1[0]) + _dot(2.0 * p2 - x, w2[0])
         + b1[...])
    h_ref[...] = h
    y_ref[...] = h * nrm


def _combine2_body(h_ref, t1a, t1b, t2a, t2b, d0, d1,
                   w0, w1, w2, b2, wfc, bfc, o_ref):
    nrm = _norm_col(d0, d1)
    h = h_ref[...]
    p1 = (t1a[0] + t1b[0]) * nrm
    p2 = (t2a[0] + t2b[0]) * nrm
    h2 = (_dot(h, w0[0]) - _dot(p1, w1[0]) + _dot(2.0 * p2 - h, w2[0])
          + b2[...])
    o_ref[...] = _dot(h2, wfc[0]) + bfc[...]


def _row_spec(bn, d):
    return pl.BlockSpec((bn, d), lambda i: (i, 0))


def _half_spec(bn, h, which):
    return pl.BlockSpec((1, bn, h), lambda i, _w=which: (_w, i, 0))


def _w_spec(k):
    return pl.BlockSpec((1, 128, 128), lambda i, _k=k: (_k, 0, 0))


def _bias_spec():
    return pl.BlockSpec((1, 128), lambda i: (0, 0))


def kernel(x, edge_index, W1, b1, W2, b2, Wfc, bfc):
    N, D = x.shape
    E = edge_index.shape[1]
    src = edge_index[0]
    dst = edge_index[1]

    nch = (E // (NC * NT)) // CH
    srcidx = src.reshape(NC, NT, nch, CH)
    dstidx = dst.reshape(NC, NT, nch, CH)

    NP = (N + 8 * NT - 1) // (8 * NT) * (8 * NT)
    zerosD = jnp.zeros((NP, D), jnp.float32)
    onesD = jnp.ones((CH, D), jnp.float32)

    degp = _deg_sc(dstidx, zerosD, onesD)

    grid = (N // BN,)
    dspecs = [_half_spec(BN, D, 0), _half_spec(BN, D, 1)]

    y1 = pl.pallas_call(
        _prep_body, grid=grid,
        in_specs=[_row_spec(BN, D)] + dspecs,
        out_specs=_row_spec(BN, D),
        out_shape=jax.ShapeDtypeStruct((N, D), jnp.float32),
    )(x, degp, degp)

    def seg(table):
        return _segsum_sc(D, table, srcidx, dstidx, zerosD)

    def zscale(sarr):
        return pl.pallas_call(
            _zscale_body, grid=grid,
            in_specs=[_half_spec(BN, D, 0), _half_spec(BN, D, 1)] + dspecs,
            out_specs=_row_spec(BN, D),
            out_shape=jax.ShapeDtypeStruct((N, D), jnp.float32),
        )(sarr, sarr, degp, degp)

    s1 = seg(y1)
    z1 = zscale(s1)
    s2 = seg(z1)

    h1, y2 = pl.pallas_call(
        _combine1_body, grid=grid,
        in_specs=([_row_spec(BN, D),
                   _half_spec(BN, D, 0), _half_spec(BN, D, 1),
                   _half_spec(BN, D, 0), _half_spec(BN, D, 1)]
                  + dspecs
                  + [_w_spec(0), _w_spec(1), _w_spec(2), _bias_spec()]),
        out_specs=[_row_spec(BN, D), _row_spec(BN, D)],
        out_shape=[jax.ShapeDtypeStruct((N, D), jnp.float32),
                   jax.ShapeDtypeStruct((N, D), jnp.float32)],
    )(x, s1, s1, s2, s2, degp, degp, W1, W1, W1, b1.reshape(1, D))

    t1 = seg(y2)
    z2 = zscale(t1)
    t2 = seg(z2)

    out = pl.pallas_call(
        _combine2_body, grid=grid,
        in_specs=([_row_spec(BN, D),
                   _half_spec(BN, D, 0), _half_spec(BN, D, 1),
                   _half_spec(BN, D, 0), _half_spec(BN, D, 1)]
                  + dspecs
                  + [_w_spec(0), _w_spec(1), _w_spec(2), _bias_spec(),
                     pl.BlockSpec((1, 128, 128), lambda i: (0, 0, 0)),
                     _bias_spec()]),
        out_specs=_row_spec(BN, D),
        out_shape=jax.ShapeDtypeStruct((N, D), jnp.float32),
    )(h1, t1, t1, t2, t2, degp, degp, W2, W2, W2, b2.reshape(1, D),
      Wfc.reshape(1, D, D), bfc.reshape(1, D))

    return out


# speedup vs baseline: 13.2136x; 1.4248x over previous
import functools

import jax
import jax.numpy as jnp
from jax import lax
from jax.experimental import pallas as pl
from jax.experimental.pallas import tpu as pltpu
from jax.experimental.pallas import tpu_sc as plsc

NC = 2
NT = 16
CH = 125
BN = 1000


def _segsum_sc(D, table, srcidx, dstidx, zeros):
    nch = srcidx.shape[2]
    NP = zeros.shape[0]
    rpt = NP // NT
    NBUF = 2
    NPH = 2
    hc = nch // NPH
    mesh = plsc.VectorSubcoreMesh(core_axis_name="c", subcore_axis_name="s")

    @functools.partial(
        pl.kernel,
        mesh=mesh,
        out_type=jax.ShapeDtypeStruct((NC, NP, D), jnp.float32),
        scratch_types=[
            pltpu.VMEM((hc, CH), jnp.int32),
            pltpu.VMEM((hc, CH), jnp.int32),
            pltpu.VMEM((NBUF, CH, D), jnp.float32),
            pltpu.VMEM_SHARED((NP, D), jnp.float32),
        ] + [pltpu.SemaphoreType.DMA] * NBUF,
    )
    def seg(table_hbm, src_hbm, dst_hbm, zeros_hbm, out_hbm,
            sidx, didx, gbuf, acc, g0, g1):
        c = lax.axis_index("c")
        s = lax.axis_index("s")
        gsems = (g0, g1)

        @pl.when(s == 0)
        def _():
            pltpu.sync_copy(zeros_hbm, acc)

        plsc.subcore_barrier()

        for ph in range(NPH):
            pltpu.sync_copy(src_hbm.at[c, s, pl.ds(ph * hc, hc)], sidx)
            pltpu.sync_copy(dst_hbm.at[c, s, pl.ds(ph * hc, hc)], didx)
            for b in range(NBUF):
                pltpu.async_copy(table_hbm.at[sidx.at[b]], gbuf.at[b],
                                 gsems[b])

            def body(g, carry):
                for b in range(NBUF):
                    j = g * NBUF + b
                    pltpu.make_async_copy(table_hbm.at[sidx.at[j]],
                                          gbuf.at[b], gsems[b]).wait()
                    pltpu.sync_copy(gbuf.at[b], acc.at[didx.at[j]], add=True)
                    jn = j + NBUF

                    @pl.when(jn < hc)
                    def _():
                        pltpu.async_copy(table_hbm.at[sidx.at[jn]],
                                         gbuf.at[b], gsems[b])
                return carry

            lax.fori_loop(0, hc // NBUF, body, 0)
        plsc.subcore_barrier()
        pltpu.sync_copy(acc.at[pl.ds(s * rpt, rpt)],
                        out_hbm.at[c, pl.ds(s * rpt, rpt)])

    return seg(table, srcidx, dstidx, zeros)


def _deg_sc(dstidx, zerosD, onesD):
    nch = dstidx.shape[2]
    NP, D = zerosD.shape
    rpt = NP // NT
    mesh = plsc.VectorSubcoreMesh(core_axis_name="c", subcore_axis_name="s")

    @functools.partial(
        pl.kernel,
        mesh=mesh,
        out_type=jax.ShapeDtypeStruct((NC, NP, D), jnp.float32),
        scratch_types=[
            pltpu.VMEM((nch, CH), jnp.int32),
            pltpu.VMEM((CH, D), jnp.float32),
            pltpu.VMEM_SHARED((NP, D), jnp.float32),
        ],
    )
    def deg(dst_hbm, zeros_hbm, ones_hbm, out_hbm, didx, onesb, acc):
        c = lax.axis_index("c")
        s = lax.axis_index("s")
        pltpu.sync_copy(dst_hbm.at[c, s], didx)
        pltpu.sync_copy(ones_hbm, onesb)

        @pl.when(s == 0)
        def _():
            pltpu.sync_copy(zeros_hbm, acc)

        plsc.subcore_barrier()

        def body(j, carry):
            pltpu.sync_copy(onesb, acc.at[didx.at[j]], add=True)
            return carry

        lax.fori_loop(0, nch, body, 0)
        plsc.subcore_barrier()
        pltpu.sync_copy(acc.at[pl.ds(s * rpt, rpt)],
                        out_hbm.at[c, pl.ds(s * rpt, rpt)])

    return deg(dstidx, zerosD, onesD)


def _norm_col(d0, d1):
    deg = d0[0, :, 0:1] + d1[0, :, 0:1]
    return lax.rsqrt(jnp.maximum(deg, 1.0))


def _prep_body(x_ref, d0, d1, y_ref):
    y_ref[...] = x_ref[...] * _norm_col(d0, d1)


def _zscale_body(sa, sb, d0, d1, z_ref):
    deg = d0[0, :, 0:1] + d1[0, :, 0:1]
    dinv = 1.0 / jnp.maximum(deg, 1.0)
    z_ref[...] = (sa[0] + sb[0]) * dinv


def _dot(a, b):
    return jnp.dot(a, b, preferred_element_type=jnp.float32)


def _combine1_body(x_ref, s1a, s1b, s2a, s2b, d0, d1,
                   w0, w1, w2, b1, h_ref, y_ref):
    nrm = _norm_col(d0, d1)
    x = x_ref[...]
    p1 = (s1a[0] + s1b[0]) * nrm
    p2 = (s2a[0] + s2b[0]) * nrm
    h = (_dot(x, w0[0]) - _dot(p1, w1[0]) + _dot(2.0 * p2 - x, w2[0])
         + b1[...])
    h_ref[...] = h
    y_ref[...] = h * nrm


def _combine2_body(h_ref, t1a, t1b, t2a, t2b, d0, d1,
                   w0, w1, w2, b2, wfc, bfc, o_ref):
    nrm = _norm_col(d0, d1)
    h = h_ref[...]
    p1 = (t1a[0] + t1b[0]) * nrm
    p2 = (t2a[0] + t2b[0]) * nrm
    h2 = (_dot(h, w0[0]) - _dot(p1, w1[0]) + _dot(2.0 * p2 - h, w2[0])
          + b2[...])
    o_ref[...] = _dot(h2, wfc[0]) + bfc[...]


def _row_spec(bn, d):
    return pl.BlockSpec((bn, d), lambda i: (i, 0))


def _half_spec(bn, h, which):
    return pl.BlockSpec((1, bn, h), lambda i, _w=which: (_w, i, 0))


def _w_spec(k):
    return pl.BlockSpec((1, 128, 128), lambda i, _k=k: (_k, 0, 0))


def _bias_spec():
    return pl.BlockSpec((1, 128), lambda i: (0, 0))


def kernel(x, edge_index, W1, b1, W2, b2, Wfc, bfc):
    N, D = x.shape
    E = edge_index.shape[1]
    src = edge_index[0]
    dst = edge_index[1]

    nch = (E // (NC * NT)) // CH
    srcidx = src.reshape(NC, NT, nch, CH)
    dstidx = dst.reshape(NC, NT, nch, CH)

    NP = (N + 8 * NT - 1) // (8 * NT) * (8 * NT)
    zerosD = jnp.zeros((NP, D), jnp.float32)
    onesD = jnp.ones((CH, D), jnp.float32)

    degp = _deg_sc(dstidx, zerosD, onesD)

    grid = (N // BN,)
    dspecs = [_half_spec(BN, D, 0), _half_spec(BN, D, 1)]

    y1 = pl.pallas_call(
        _prep_body, grid=grid,
        in_specs=[_row_spec(BN, D)] + dspecs,
        out_specs=_row_spec(BN, D),
        out_shape=jax.ShapeDtypeStruct((N, D), jnp.float32),
    )(x, degp, degp)

    def seg(table):
        return _segsum_sc(D, table, srcidx, dstidx, zerosD)

    def zscale(sarr):
        return pl.pallas_call(
            _zscale_body, grid=grid,
            in_specs=[_half_spec(BN, D, 0), _half_spec(BN, D, 1)] + dspecs,
            out_specs=_row_spec(BN, D),
            out_shape=jax.ShapeDtypeStruct((N, D), jnp.float32),
        )(sarr, sarr, degp, degp)

    s1 = seg(y1)
    z1 = zscale(s1)
    s2 = seg(z1)

    h1, y2 = pl.pallas_call(
        _combine1_body, grid=grid,
        in_specs=([_row_spec(BN, D),
                   _half_spec(BN, D, 0), _half_spec(BN, D, 1),
                   _half_spec(BN, D, 0), _half_spec(BN, D, 1)]
                  + dspecs
                  + [_w_spec(0), _w_spec(1), _w_spec(2), _bias_spec()]),
        out_specs=[_row_spec(BN, D), _row_spec(BN, D)],
        out_shape=[jax.ShapeDtypeStruct((N, D), jnp.float32),
                   jax.ShapeDtypeStruct((N, D), jnp.float32)],
    )(x, s1, s1, s2, s2, degp, degp, W1, W1, W1, b1.reshape(1, D))

    t1 = seg(y2)
    z2 = zscale(t1)
    t2 = seg(z2)

    out = pl.pallas_call(
        _combine2_body, grid=grid,
        in_specs=([_row_spec(BN, D),
                   _half_spec(BN, D, 0), _half_spec(BN, D, 1),
                   _half_spec(BN, D, 0), _half_spec(BN, D, 1)]
                  + dspecs
                  + [_w_spec(0), _w_spec(1), _w_spec(2), _bias_spec(),
                     pl.BlockSpec((1, 128, 128), lambda i: (0, 0, 0)),
                     _bias_spec()]),
        out_specs=_row_spec(BN, D),
        out_shape=jax.ShapeDtypeStruct((N, D), jnp.float32),
    )(h1, t1, t1, t2, t2, degp, degp, W2, W2, W2, b2.reshape(1, D),
      Wfc.reshape(1, D, D), bfc.reshape(1, D))

    return out


# speedup vs baseline: 13.4887x; 1.0208x over previous
import functools

import jax
import jax.numpy as jnp
from jax import lax
from jax.experimental import pallas as pl
from jax.experimental.pallas import tpu as pltpu
from jax.experimental.pallas import tpu_sc as plsc

NC = 2
NT = 16
CH = 125
BN = 1000


def _segsum_sc(D, table, eidx, zeros):
    nch = eidx.shape[3]
    NP = zeros.shape[0]
    rpt = NP // NT
    NBUF = 2
    NPH = 2
    hc = nch // NPH
    mesh = plsc.VectorSubcoreMesh(core_axis_name="c", subcore_axis_name="s")

    @functools.partial(
        pl.kernel,
        mesh=mesh,
        out_type=jax.ShapeDtypeStruct((NC, NP, D), jnp.float32),
        scratch_types=[
            pltpu.VMEM((hc, CH), jnp.int32),
            pltpu.VMEM((hc, CH), jnp.int32),
            pltpu.VMEM((NBUF, CH, D), jnp.float32),
            pltpu.VMEM_SHARED((NP, D), jnp.float32),
        ] + [pltpu.SemaphoreType.DMA] * NBUF,
    )
    def seg(table_hbm, eidx_hbm, zeros_hbm, out_hbm,
            sidx, didx, gbuf, acc, g0, g1):
        c = lax.axis_index("c")
        s = lax.axis_index("s")
        gsems = (g0, g1)
        pltpu.sync_copy(zeros_hbm.at[pl.ds(s * rpt, rpt)],
                        acc.at[pl.ds(s * rpt, rpt)])
        plsc.subcore_barrier()

        for ph in range(NPH):
            pltpu.sync_copy(eidx_hbm.at[0, c, s, pl.ds(ph * hc, hc)], sidx)
            pltpu.sync_copy(eidx_hbm.at[1, c, s, pl.ds(ph * hc, hc)], didx)
            for b in range(NBUF):
                pltpu.async_copy(table_hbm.at[sidx.at[b]], gbuf.at[b],
                                 gsems[b])

            def body(g, carry):
                for b in range(NBUF):
                    j = g * NBUF + b
                    pltpu.make_async_copy(table_hbm.at[sidx.at[j]],
                                          gbuf.at[b], gsems[b]).wait()
                    pltpu.sync_copy(gbuf.at[b], acc.at[didx.at[j]], add=True)
                    jn = j + NBUF

                    @pl.when(jn < hc)
                    def _():
                        pltpu.async_copy(table_hbm.at[sidx.at[jn]],
                                         gbuf.at[b], gsems[b])
                return carry

            lax.fori_loop(0, hc // NBUF, body, 0)
        plsc.subcore_barrier()
        pltpu.sync_copy(acc.at[pl.ds(s * rpt, rpt)],
                        out_hbm.at[c, pl.ds(s * rpt, rpt)])

    return seg(table, eidx, zeros)


def _deg_sc(eidx, zerosD, onesD):
    nch = eidx.shape[3]
    NP, D = zerosD.shape
    rpt = NP // NT
    mesh = plsc.VectorSubcoreMesh(core_axis_name="c", subcore_axis_name="s")

    @functools.partial(
        pl.kernel,
        mesh=mesh,
        out_type=jax.ShapeDtypeStruct((NC, NP, D), jnp.float32),
        scratch_types=[
            pltpu.VMEM((nch, CH), jnp.int32),
            pltpu.VMEM((CH, D), jnp.float32),
            pltpu.VMEM_SHARED((NP, D), jnp.float32),
        ],
    )
    def deg(eidx_hbm, zeros_hbm, ones_hbm, out_hbm, didx, onesb, acc):
        c = lax.axis_index("c")
        s = lax.axis_index("s")
        pltpu.sync_copy(eidx_hbm.at[1, c, s], didx)
        pltpu.sync_copy(ones_hbm, onesb)
        pltpu.sync_copy(zeros_hbm.at[pl.ds(s * rpt, rpt)],
                        acc.at[pl.ds(s * rpt, rpt)])
        plsc.subcore_barrier()

        def body(j, carry):
            pltpu.sync_copy(onesb, acc.at[didx.at[j]], add=True)
            return carry

        lax.fori_loop(0, nch, body, 0)
        plsc.subcore_barrier()
        pltpu.sync_copy(acc.at[pl.ds(s * rpt, rpt)],
                        out_hbm.at[c, pl.ds(s * rpt, rpt)])

    return deg(eidx, zerosD, onesD)


def _norm_col(d0, d1):
    deg = (d0[0, :, 0:1] + d1[0, :, 0:1]).astype(jnp.float32)
    return lax.rsqrt(jnp.maximum(deg, 1.0))


def _prep_body(x_ref, d0, d1, y_ref, n_ref):
    nrm = _norm_col(d0, d1)
    y_ref[...] = x_ref[...] * nrm
    n_ref[...] = jnp.broadcast_to(nrm, (nrm.shape[0], n_ref.shape[1]))


def _zscale_body(sa, sb, n_ref, z_ref):
    nrm = n_ref[:, 0:1]
    z_ref[...] = (sa[0] + sb[0]) * (nrm * nrm)


def _dot(a, b):
    return jnp.dot(a, b, preferred_element_type=jnp.float32)


def _combine1_body(x_ref, s1a, s1b, s2a, s2b, n_ref,
                   w0, w1, w2, b1, h_ref, y_ref):
    nrm = n_ref[:, 0:1]
    x = x_ref[...]
    p1 = (s1a[0] + s1b[0]) * nrm
    p2 = (s2a[0] + s2b[0]) * nrm
    h = (_dot(x, w0[0]) - _dot(p1, w1[0]) + _dot(2.0 * p2 - x, w2[0])
         + b1[...])
    h_ref[...] = h
    y_ref[...] = h * nrm


def _combine2_body(h_ref, t1a, t1b, t2a, t2b, n_ref,
                   w0, w1, w2, b2, wfc, bfc, o_ref):
    nrm = n_ref[:, 0:1]
    h = h_ref[...]
    p1 = (t1a[0] + t1b[0]) * nrm
    p2 = (t2a[0] + t2b[0]) * nrm
    h2 = (_dot(h, w0[0]) - _dot(p1, w1[0]) + _dot(2.0 * p2 - h, w2[0])
          + b2[...])
    o_ref[...] = _dot(h2, wfc[0]) + bfc[...]


def _row_spec(bn, d):
    return pl.BlockSpec((bn, d), lambda i: (i, 0))


def _half_spec(bn, h, which):
    return pl.BlockSpec((1, bn, h), lambda i, _w=which: (_w, i, 0))


def _w_spec(k):
    return pl.BlockSpec((1, 128, 128), lambda i, _k=k: (_k, 0, 0))


def _bias_spec():
    return pl.BlockSpec((1, 128), lambda i: (0, 0))


def kernel(x, edge_index, W1, b1, W2, b2, Wfc, bfc):
    N, D = x.shape
    E = edge_index.shape[1]
    src = edge_index[0]
    dst = edge_index[1]

    nch = (E // (NC * NT)) // CH
    eidx = edge_index.reshape(2, NC, NT, nch, CH)

    NP = (N + 8 * NT - 1) // (8 * NT) * (8 * NT)
    zerosD = jnp.zeros((NP, D), jnp.float32)
    onesD = jnp.ones((CH, D), jnp.float32)

    degp = _deg_sc(eidx, zerosD, onesD)

    grid = (N // BN,)
    dspecs = [_half_spec(BN, D, 0), _half_spec(BN, D, 1)]
    NW = 8

    y1, nrm = pl.pallas_call(
        _prep_body, grid=grid,
        in_specs=[_row_spec(BN, D)] + dspecs,
        out_specs=[_row_spec(BN, D), _row_spec(BN, NW)],
        out_shape=[jax.ShapeDtypeStruct((N, D), jnp.float32),
                   jax.ShapeDtypeStruct((N, NW), jnp.float32)],
    )(x, degp, degp)

    def seg(table):
        return _segsum_sc(D, table, eidx, zerosD)

    def zscale(sarr):
        return pl.pallas_call(
            _zscale_body, grid=grid,
            in_specs=[_half_spec(BN, D, 0), _half_spec(BN, D, 1),
                      _row_spec(BN, NW)],
            out_specs=_row_spec(BN, D),
            out_shape=jax.ShapeDtypeStruct((N, D), jnp.float32),
        )(sarr, sarr, nrm)

    s1 = seg(y1)
    z1 = zscale(s1)
    s2 = seg(z1)

    h1, y2 = pl.pallas_call(
        _combine1_body, grid=grid,
        in_specs=([_row_spec(BN, D),
                   _half_spec(BN, D, 0), _half_spec(BN, D, 1),
                   _half_spec(BN, D, 0), _half_spec(BN, D, 1),
                   _row_spec(BN, NW)]
                  + [_w_spec(0), _w_spec(1), _w_spec(2), _bias_spec()]),
        out_specs=[_row_spec(BN, D), _row_spec(BN, D)],
        out_shape=[jax.ShapeDtypeStruct((N, D), jnp.float32),
                   jax.ShapeDtypeStruct((N, D), jnp.float32)],
    )(x, s1, s1, s2, s2, nrm, W1, W1, W1, b1.reshape(1, D))

    t1 = seg(y2)
    z2 = zscale(t1)
    t2 = seg(z2)

    out = pl.pallas_call(
        _combine2_body, grid=grid,
        in_specs=([_row_spec(BN, D),
                   _half_spec(BN, D, 0), _half_spec(BN, D, 1),
                   _half_spec(BN, D, 0), _half_spec(BN, D, 1),
                   _row_spec(BN, NW)]
                  + [_w_spec(0), _w_spec(1), _w_spec(2), _bias_spec(),
                     pl.BlockSpec((1, 128, 128), lambda i: (0, 0, 0)),
                     _bias_spec()]),
        out_specs=_row_spec(BN, D),
        out_shape=jax.ShapeDtypeStruct((N, D), jnp.float32),
    )(h1, t1, t1, t2, t2, nrm, W2, W2, W2, b2.reshape(1, D),
      Wfc.reshape(1, D, D), bfc.reshape(1, D))

    return out


# speedup vs baseline: 13.7862x; 1.0221x over previous
import functools

import jax
import jax.numpy as jnp
from jax import lax
from jax.experimental import pallas as pl
from jax.experimental.pallas import tpu as pltpu
from jax.experimental.pallas import tpu_sc as plsc

NC = 2
NT = 16
CH = 125
BN = 2000


def _segsum_sc(D, table, eidx, zeros):
    nch = eidx.shape[3]
    NP = zeros.shape[0]
    rpt = NP // NT
    NBUF = 2
    NPH = 2
    hc = nch // NPH
    mesh = plsc.VectorSubcoreMesh(core_axis_name="c", subcore_axis_name="s")

    @functools.partial(
        pl.kernel,
        mesh=mesh,
        out_type=jax.ShapeDtypeStruct((NC, NP, D), jnp.float32),
        scratch_types=[
            pltpu.VMEM((hc, CH), jnp.int32),
            pltpu.VMEM((hc, CH), jnp.int32),
            pltpu.VMEM((NBUF, CH, D), jnp.float32),
            pltpu.VMEM_SHARED((NP, D), jnp.float32),
        ] + [pltpu.SemaphoreType.DMA] * NBUF,
    )
    def seg(table_hbm, eidx_hbm, zeros_hbm, out_hbm,
            sidx, didx, gbuf, acc, g0, g1):
        c = lax.axis_index("c")
        s = lax.axis_index("s")
        gsems = (g0, g1)
        pltpu.sync_copy(zeros_hbm.at[pl.ds(s * rpt, rpt)],
                        acc.at[pl.ds(s * rpt, rpt)])
        plsc.subcore_barrier()

        for ph in range(NPH):
            pltpu.sync_copy(eidx_hbm.at[0, c, s, pl.ds(ph * hc, hc)], sidx)
            pltpu.sync_copy(eidx_hbm.at[1, c, s, pl.ds(ph * hc, hc)], didx)
            for b in range(NBUF):
                pltpu.async_copy(table_hbm.at[sidx.at[b]], gbuf.at[b],
                                 gsems[b])

            def body(g, carry):
                for b in range(NBUF):
                    j = g * NBUF + b
                    pltpu.make_async_copy(table_hbm.at[sidx.at[j]],
                                          gbuf.at[b], gsems[b]).wait()
                    pltpu.sync_copy(gbuf.at[b], acc.at[didx.at[j]], add=True)
                    jn = j + NBUF

                    @pl.when(jn < hc)
                    def _():
                        pltpu.async_copy(table_hbm.at[sidx.at[jn]],
                                         gbuf.at[b], gsems[b])
                return carry

            lax.fori_loop(0, hc // NBUF, body, 0)
        plsc.subcore_barrier()
        pltpu.sync_copy(acc.at[pl.ds(s * rpt, rpt)],
                        out_hbm.at[c, pl.ds(s * rpt, rpt)])

    return seg(table, eidx, zeros)


def _deg_sc(eidx, zerosD, onesD):
    nch = eidx.shape[3]
    NP, D = zerosD.shape
    rpt = NP // NT
    mesh = plsc.VectorSubcoreMesh(core_axis_name="c", subcore_axis_name="s")

    @functools.partial(
        pl.kernel,
        mesh=mesh,
        out_type=jax.ShapeDtypeStruct((NC, NP, D), jnp.float32),
        scratch_types=[
            pltpu.VMEM((nch, CH), jnp.int32),
            pltpu.VMEM((CH, D), jnp.float32),
            pltpu.VMEM_SHARED((NP, D), jnp.float32),
        ],
    )
    def deg(eidx_hbm, zeros_hbm, ones_hbm, out_hbm, didx, onesb, acc):
        c = lax.axis_index("c")
        s = lax.axis_index("s")
        pltpu.sync_copy(eidx_hbm.at[1, c, s], didx)
        pltpu.sync_copy(ones_hbm, onesb)
        pltpu.sync_copy(zeros_hbm.at[pl.ds(s * rpt, rpt)],
                        acc.at[pl.ds(s * rpt, rpt)])
        plsc.subcore_barrier()

        def body(j, carry):
            pltpu.sync_copy(onesb, acc.at[didx.at[j]], add=True)
            return carry

        lax.fori_loop(0, nch, body, 0)
        plsc.subcore_barrier()
        pltpu.sync_copy(acc.at[pl.ds(s * rpt, rpt)],
                        out_hbm.at[c, pl.ds(s * rpt, rpt)])

    return deg(eidx, zerosD, onesD)


def _norm_col(d0, d1):
    deg = (d0[0, :, 0:1] + d1[0, :, 0:1]).astype(jnp.float32)
    return lax.rsqrt(jnp.maximum(deg, 1.0))


def _prep_body(x_ref, d0, d1, y_ref, n_ref):
    nrm = _norm_col(d0, d1)
    y_ref[...] = x_ref[...] * nrm
    n_ref[...] = jnp.broadcast_to(nrm, (nrm.shape[0], n_ref.shape[1]))


def _zscale_body(sa, sb, n_ref, z_ref):
    nrm = n_ref[:, 0:1]
    z_ref[...] = (sa[0] + sb[0]) * (nrm * nrm)


def _dot(a, b):
    return jnp.dot(a, b, preferred_element_type=jnp.float32)


def _combine1_body(x_ref, s1a, s1b, s2a, s2b, n_ref,
                   w0, w1, w2, b1, h_ref, y_ref):
    nrm = n_ref[:, 0:1]
    x = x_ref[...]
    p1 = (s1a[0] + s1b[0]) * nrm
    p2 = (s2a[0] + s2b[0]) * nrm
    h = (_dot(x, w0[0]) - _dot(p1, w1[0]) + _dot(2.0 * p2 - x, w2[0])
         + b1[...])
    h_ref[...] = h
    y_ref[...] = h * nrm


def _combine2_body(h_ref, t1a, t1b, t2a, t2b, n_ref,
                   w0, w1, w2, b2, wfc, bfc, o_ref):
    nrm = n_ref[:, 0:1]
    h = h_ref[...]
    p1 = (t1a[0] + t1b[0]) * nrm
    p2 = (t2a[0] + t2b[0]) * nrm
    h2 = (_dot(h, w0[0]) - _dot(p1, w1[0]) + _dot(2.0 * p2 - h, w2[0])
          + b2[...])
    o_ref[...] = _dot(h2, wfc[0]) + bfc[...]


def _row_spec(bn, d):
    return pl.BlockSpec((bn, d), lambda i: (i, 0))


def _half_spec(bn, h, which):
    return pl.BlockSpec((1, bn, h), lambda i, _w=which: (_w, i, 0))


def _w_spec(k):
    return pl.BlockSpec((1, 128, 128), lambda i, _k=k: (_k, 0, 0))


def _bias_spec():
    return pl.BlockSpec((1, 128), lambda i: (0, 0))


def kernel(x, edge_index, W1, b1, W2, b2, Wfc, bfc):
    N, D = x.shape
    E = edge_index.shape[1]
    src = edge_index[0]
    dst = edge_index[1]

    nch = (E // (NC * NT)) // CH
    eidx = edge_index.reshape(2, NC, NT, nch, CH)

    NP = (N + 8 * NT - 1) // (8 * NT) * (8 * NT)
    zerosD = jnp.zeros((NP, D), jnp.float32)
    onesD = jnp.ones((CH, D), jnp.float32)

    degp = _deg_sc(eidx, zerosD, onesD)

    grid = (N // BN,)
    dspecs = [_half_spec(BN, D, 0), _half_spec(BN, D, 1)]
    NW = 8

    y1, nrm = pl.pallas_call(
        _prep_body, grid=grid,
        in_specs=[_row_spec(BN, D)] + dspecs,
        out_specs=[_row_spec(BN, D), _row_spec(BN, NW)],
        out_shape=[jax.ShapeDtypeStruct((N, D), jnp.float32),
                   jax.ShapeDtypeStruct((N, NW), jnp.float32)],
    )(x, degp, degp)

    def seg(table):
        return _segsum_sc(D, table, eidx, zerosD)

    def zscale(sarr):
        return pl.pallas_call(
            _zscale_body, grid=grid,
            in_specs=[_half_spec(BN, D, 0), _half_spec(BN, D, 1),
                      _row_spec(BN, NW)],
            out_specs=_row_spec(BN, D),
            out_shape=jax.ShapeDtypeStruct((N, D), jnp.float32),
        )(sarr, sarr, nrm)

    s1 = seg(y1)
    z1 = zscale(s1)
    s2 = seg(z1)

    h1, y2 = pl.pallas_call(
        _combine1_body, grid=grid,
        in_specs=([_row_spec(BN, D),
                   _half_spec(BN, D, 0), _half_spec(BN, D, 1),
                   _half_spec(BN, D, 0), _half_spec(BN, D, 1),
                   _row_spec(BN, NW)]
                  + [_w_spec(0), _w_spec(1), _w_spec(2), _bias_spec()]),
        out_specs=[_row_spec(BN, D), _row_spec(BN, D)],
        out_shape=[jax.ShapeDtypeStruct((N, D), jnp.float32),
                   jax.ShapeDtypeStruct((N, D), jnp.float32)],
    )(x, s1, s1, s2, s2, nrm, W1, W1, W1, b1.reshape(1, D))

    t1 = seg(y2)
    z2 = zscale(t1)
    t2 = seg(z2)

    out = pl.pallas_call(
        _combine2_body, grid=grid,
        in_specs=([_row_spec(BN, D),
                   _half_spec(BN, D, 0), _half_spec(BN, D, 1),
                   _half_spec(BN, D, 0), _half_spec(BN, D, 1),
                   _row_spec(BN, NW)]
                  + [_w_spec(0), _w_spec(1), _w_spec(2), _bias_spec(),
                     pl.BlockSpec((1, 128, 128), lambda i: (0, 0, 0)),
                     _bias_spec()]),
        out_specs=_row_spec(BN, D),
        out_shape=jax.ShapeDtypeStruct((N, D), jnp.float32),
    )(h1, t1, t1, t2, t2, nrm, W2, W2, W2, b2.reshape(1, D),
      Wfc.reshape(1, D, D), bfc.reshape(1, D))

    return out


# speedup vs baseline: 13.8156x; 1.0021x over previous
import functools

import jax
import jax.numpy as jnp
from jax import lax
from jax.experimental import pallas as pl
from jax.experimental.pallas import tpu as pltpu
from jax.experimental.pallas import tpu_sc as plsc

NC = 2
NT = 16
CH = 100
BN = 2000
NPH = 4


def _segsum_sc(D, table, eidx, zeros):
    hc = eidx.shape[4]
    NP = zeros.shape[0]
    rpt = NP // NT
    NBUF = 3
    ngrp = (hc - 1) // NBUF
    mesh = plsc.VectorSubcoreMesh(core_axis_name="c", subcore_axis_name="s")

    @functools.partial(
        pl.kernel,
        mesh=mesh,
        out_type=jax.ShapeDtypeStruct((NC, NP, D), jnp.float32),
        scratch_types=[
            pltpu.VMEM((hc, CH), jnp.int32),
            pltpu.VMEM((hc, CH), jnp.int32),
            pltpu.VMEM((NBUF, CH, D), jnp.float32),
            pltpu.VMEM_SHARED((NP, D), jnp.float32),
        ] + [pltpu.SemaphoreType.DMA] * (2 * NBUF),
    )
    def seg(table_hbm, eidx_hbm, zeros_hbm, out_hbm,
            sidx, didx, gbuf, acc, g0, g1, g2, s0, s1_, s2_):
        c = lax.axis_index("c")
        s = lax.axis_index("s")
        gsems = (g0, g1, g2)
        ssems = (s0, s1_, s2_)
        pltpu.sync_copy(zeros_hbm.at[pl.ds(s * rpt, rpt)],
                        acc.at[pl.ds(s * rpt, rpt)])
        plsc.subcore_barrier()

        def wait_g(b, j):
            pltpu.make_async_copy(table_hbm.at[sidx.at[j]],
                                  gbuf.at[b], gsems[b]).wait()

        def wait_s(b):
            pltpu.make_async_copy(gbuf.at[b], acc.at[didx.at[0]],
                                  ssems[b]).wait()

        for ph in range(NPH):
            pltpu.sync_copy(eidx_hbm.at[0, c, s, ph], sidx)
            pltpu.sync_copy(eidx_hbm.at[1, c, s, ph], didx)
            for b in range(2):
                pltpu.async_copy(table_hbm.at[sidx.at[b]], gbuf.at[b],
                                 gsems[b])

            def body(g, carry):
                for t in range(NBUF):
                    j = g * NBUF + t
                    wait_g(t, j)
                    pltpu.async_copy(gbuf.at[t], acc.at[didx.at[j]],
                                     ssems[t], add=True)
                    jn = j + 2
                    bn = (t + 2) % NBUF

                    @pl.when(jn < hc)
                    def _():
                        if t == 0:
                            @pl.when(g > 0)
                            def _():
                                wait_s(bn)
                        else:
                            wait_s(bn)
                        pltpu.async_copy(table_hbm.at[sidx.at[jn]],
                                         gbuf.at[bn], gsems[bn])
                return carry

            lax.fori_loop(0, ngrp, body, 0)
            wait_g(0, hc - 1)
            pltpu.sync_copy(gbuf.at[0], acc.at[didx.at[hc - 1]], add=True)
            wait_s(1)
            wait_s(2)
        plsc.subcore_barrier()
        pltpu.sync_copy(acc.at[pl.ds(s * rpt, rpt)],
                        out_hbm.at[c, pl.ds(s * rpt, rpt)])

    return seg(table, eidx, zeros)


def _deg_sc(eidx, zerosD, onesD):
    hc = eidx.shape[4]
    NP, D = zerosD.shape
    rpt = NP // NT
    mesh = plsc.VectorSubcoreMesh(core_axis_name="c", subcore_axis_name="s")

    @functools.partial(
        pl.kernel,
        mesh=mesh,
        out_type=jax.ShapeDtypeStruct((NC, NP, D), jnp.float32),
        scratch_types=[
            pltpu.VMEM((NPH, hc, CH), jnp.int32),
            pltpu.VMEM((CH, D), jnp.float32),
            pltpu.VMEM_SHARED((NP, D), jnp.float32),
        ],
    )
    def deg(eidx_hbm, zeros_hbm, ones_hbm, out_hbm, didx, onesb, acc):
        c = lax.axis_index("c")
        s = lax.axis_index("s")
        pltpu.sync_copy(eidx_hbm.at[1, c, s], didx)
        pltpu.sync_copy(ones_hbm, onesb)
        pltpu.sync_copy(zeros_hbm.at[pl.ds(s * rpt, rpt)],
                        acc.at[pl.ds(s * rpt, rpt)])
        plsc.subcore_barrier()

        for ph in range(NPH):
            def body(j, carry, _ph=ph):
                pltpu.sync_copy(onesb, acc.at[didx.at[_ph, j]], add=True)
                return carry

            lax.fori_loop(0, hc, body, 0)
        plsc.subcore_barrier()
        pltpu.sync_copy(acc.at[pl.ds(s * rpt, rpt)],
                        out_hbm.at[c, pl.ds(s * rpt, rpt)])

    return deg(eidx, zerosD, onesD)


def _norm_col(d0, d1):
    deg = (d0[0, :, 0:1] + d1[0, :, 0:1]).astype(jnp.float32)
    return lax.rsqrt(jnp.maximum(deg, 1.0))


def _prep_body(x_ref, d0, d1, y_ref, n_ref):
    nrm = _norm_col(d0, d1)
    y_ref[...] = x_ref[...] * nrm
    n_ref[...] = jnp.broadcast_to(nrm, (nrm.shape[0], n_ref.shape[1]))


def _zscale_body(sa, sb, n_ref, z_ref):
    nrm = n_ref[:, 0:1]
    z_ref[...] = (sa[0] + sb[0]) * (nrm * nrm)


def _dot(a, b):
    return jnp.dot(a, b, preferred_element_type=jnp.float32)


def _combine1_body(x_ref, s1a, s1b, s2a, s2b, n_ref,
                   w0, w1, w2, b1, h_ref, y_ref):
    nrm = n_ref[:, 0:1]
    x = x_ref[...]
    p1 = (s1a[0] + s1b[0]) * nrm
    p2 = (s2a[0] + s2b[0]) * nrm
    h = (_dot(x, w0[0]) - _dot(p1, w1[0]) + _dot(2.0 * p2 - x, w2[0])
         + b1[...])
    h_ref[...] = h
    y_ref[...] = h * nrm


def _combine2_body(h_ref, t1a, t1b, t2a, t2b, n_ref,
                   w0, w1, w2, b2, wfc, bfc, o_ref):
    nrm = n_ref[:, 0:1]
    h = h_ref[...]
    p1 = (t1a[0] + t1b[0]) * nrm
    p2 = (t2a[0] + t2b[0]) * nrm
    h2 = (_dot(h, w0[0]) - _dot(p1, w1[0]) + _dot(2.0 * p2 - h, w2[0])
          + b2[...])
    o_ref[...] = _dot(h2, wfc[0]) + bfc[...]


def _row_spec(bn, d):
    return pl.BlockSpec((bn, d), lambda i: (i, 0))


def _half_spec(bn, h, which):
    return pl.BlockSpec((1, bn, h), lambda i, _w=which: (_w, i, 0))


def _w_spec(k):
    return pl.BlockSpec((1, 128, 128), lambda i, _k=k: (_k, 0, 0))


def _bias_spec():
    return pl.BlockSpec((1, 128), lambda i: (0, 0))


def kernel(x, edge_index, W1, b1, W2, b2, Wfc, bfc):
    N, D = x.shape
    E = edge_index.shape[1]
    src = edge_index[0]
    dst = edge_index[1]

    nch = (E // (NC * NT)) // CH
    eidx = edge_index.reshape(2, NC, NT, NPH, nch // NPH, CH)

    NP = (N + 8 * NT - 1) // (8 * NT) * (8 * NT)
    zerosD = jnp.zeros((NP, D), jnp.float32)
    onesD = jnp.ones((CH, D), jnp.float32)

    degp = _deg_sc(eidx, zerosD, onesD)

    grid = (N // BN,)
    dspecs = [_half_spec(BN, D, 0), _half_spec(BN, D, 1)]
    NW = 8

    y1, nrm = pl.pallas_call(
        _prep_body, grid=grid,
        in_specs=[_row_spec(BN, D)] + dspecs,
        out_specs=[_row_spec(BN, D), _row_spec(BN, NW)],
        out_shape=[jax.ShapeDtypeStruct((N, D), jnp.float32),
                   jax.ShapeDtypeStruct((N, NW), jnp.float32)],
    )(x, degp, degp)

    def seg(table):
        return _segsum_sc(D, table, eidx, zerosD)

    def zscale(sarr):
        return pl.pallas_call(
            _zscale_body, grid=grid,
            in_specs=[_half_spec(BN, D, 0), _half_spec(BN, D, 1),
                      _row_spec(BN, NW)],
            out_specs=_row_spec(BN, D),
            out_shape=jax.ShapeDtypeStruct((N, D), jnp.float32),
        )(sarr, sarr, nrm)

    s1 = seg(y1)
    z1 = zscale(s1)
    s2 = seg(z1)

    h1, y2 = pl.pallas_call(
        _combine1_body, grid=grid,
        in_specs=([_row_spec(BN, D),
                   _half_spec(BN, D, 0), _half_spec(BN, D, 1),
                   _half_spec(BN, D, 0), _half_spec(BN, D, 1),
                   _row_spec(BN, NW)]
                  + [_w_spec(0), _w_spec(1), _w_spec(2), _bias_spec()]),
        out_specs=[_row_spec(BN, D), _row_spec(BN, D)],
        out_shape=[jax.ShapeDtypeStruct((N, D), jnp.float32),
                   jax.ShapeDtypeStruct((N, D), jnp.float32)],
    )(x, s1, s1, s2, s2, nrm, W1, W1, W1, b1.reshape(1, D))

    t1 = seg(y2)
    z2 = zscale(t1)
    t2 = seg(z2)

    out = pl.pallas_call(
        _combine2_body, grid=grid,
        in_specs=([_row_spec(BN, D),
                   _half_spec(BN, D, 0), _half_spec(BN, D, 1),
                   _half_spec(BN, D, 0), _half_spec(BN, D, 1),
                   _row_spec(BN, NW)]
                  + [_w_spec(0), _w_spec(1), _w_spec(2), _bias_spec(),
                     pl.BlockSpec((1, 128, 128), lambda i: (0, 0, 0)),
                     _bias_spec()]),
        out_specs=_row_spec(BN, D),
        out_shape=jax.ShapeDtypeStruct((N, D), jnp.float32),
    )(h1, t1, t1, t2, t2, nrm, W2, W2, W2, b2.reshape(1, D),
      Wfc.reshape(1, D, D), bfc.reshape(1, D))

    return out


# speedup vs baseline: 13.8159x; 1.0000x over previous
import functools

import jax
import jax.numpy as jnp
from jax import lax
from jax.experimental import pallas as pl
from jax.experimental.pallas import tpu as pltpu
from jax.experimental.pallas import tpu_sc as plsc

NC = 2
NT = 16
CH = 100
BN = 2000
NPH = 4


def _segsum_sc(D, table, eidx, zeros):
    hc = eidx.shape[4]
    NP = zeros.shape[0]
    rpt = NP // NT
    NBUF = 3
    ngrp = (hc - 1) // NBUF
    mesh = plsc.VectorSubcoreMesh(core_axis_name="c", subcore_axis_name="s")

    @functools.partial(
        pl.kernel,
        mesh=mesh,
        out_type=jax.ShapeDtypeStruct((NC, NP, D), jnp.float32),
        scratch_types=[
            pltpu.VMEM((hc, CH), jnp.int32),
            pltpu.VMEM((hc, CH), jnp.int32),
            pltpu.VMEM((NBUF, CH, D), jnp.float32),
            pltpu.VMEM_SHARED((NP, D), jnp.float32),
        ] + [pltpu.SemaphoreType.DMA] * (2 * NBUF),
    )
    def seg(table_hbm, eidx_hbm, zeros_hbm, out_hbm,
            sidx, didx, gbuf, acc, g0, g1, g2, s0, s1_, s2_):
        c = lax.axis_index("c")
        s = lax.axis_index("s")
        gsems = (g0, g1, g2)
        ssems = (s0, s1_, s2_)
        pltpu.sync_copy(zeros_hbm.at[pl.ds(s * rpt, rpt)],
                        acc.at[pl.ds(s * rpt, rpt)])
        plsc.subcore_barrier()

        def wait_g(b, j):
            pltpu.make_async_copy(table_hbm.at[sidx.at[j]],
                                  gbuf.at[b], gsems[b]).wait()

        def wait_s(b):
            pltpu.make_async_copy(gbuf.at[b], acc.at[didx.at[0]],
                                  ssems[b]).wait()

        for ph in range(NPH):
            pltpu.sync_copy(eidx_hbm.at[0, c, s, ph], sidx)
            pltpu.sync_copy(eidx_hbm.at[1, c, s, ph], didx)
            for b in range(2):
                pltpu.async_copy(table_hbm.at[sidx.at[b]], gbuf.at[b],
                                 gsems[b])

            def body(g, carry):
                for t in range(NBUF):
                    j = g * NBUF + t
                    wait_g(t, j)
                    pltpu.async_copy(gbuf.at[t], acc.at[didx.at[j]],
                                     ssems[t], add=True)
                    jn = j + 2
                    bn = (t + 2) % NBUF

                    @pl.when(jn < hc)
                    def _():
                        if t == 0:
                            @pl.when(g > 0)
                            def _():
                                wait_s(bn)
                        else:
                            wait_s(bn)
                        pltpu.async_copy(table_hbm.at[sidx.at[jn]],
                                         gbuf.at[bn], gsems[bn])
                return carry

            lax.fori_loop(0, ngrp, body, 0)
            wait_g(0, hc - 1)
            pltpu.sync_copy(gbuf.at[0], acc.at[didx.at[hc - 1]], add=True)
            wait_s(1)
            wait_s(2)
        plsc.subcore_barrier()
        pltpu.sync_copy(acc.at[pl.ds(s * rpt, rpt)],
                        out_hbm.at[c, pl.ds(s * rpt, rpt)])

    return seg(table, eidx, zeros)


def _deg_sc(eidx, zerosD, onesD):
    hc = eidx.shape[4]
    NP, D = zerosD.shape
    rpt = NP // NT
    mesh = plsc.VectorSubcoreMesh(core_axis_name="c", subcore_axis_name="s")

    @functools.partial(
        pl.kernel,
        mesh=mesh,
        out_type=jax.ShapeDtypeStruct((NC, NP, D), jnp.float32),
        scratch_types=[
            pltpu.VMEM((NPH, hc, CH), jnp.int32),
            pltpu.VMEM((CH, D), jnp.float32),
            pltpu.VMEM_SHARED((NP, D), jnp.float32),
        ],
    )
    def deg(eidx_hbm, zeros_hbm, ones_hbm, out_hbm, didx, onesb, acc):
        c = lax.axis_index("c")
        s = lax.axis_index("s")
        pltpu.sync_copy(eidx_hbm.at[1, c, s], didx)
        pltpu.sync_copy(ones_hbm, onesb)
        pltpu.sync_copy(zeros_hbm.at[pl.ds(s * rpt, rpt)],
                        acc.at[pl.ds(s * rpt, rpt)])
        plsc.subcore_barrier()

        for ph in range(NPH):
            def body(j, carry, _ph=ph):
                pltpu.sync_copy(onesb, acc.at[didx.at[_ph, j]], add=True)
                return carry

            lax.fori_loop(0, hc, body, 0)
        plsc.subcore_barrier()
        pltpu.sync_copy(acc.at[pl.ds(s * rpt, rpt)],
                        out_hbm.at[c, pl.ds(s * rpt, rpt)])

    return deg(eidx, zerosD, onesD)


def _norm_col(d0, d1):
    deg = (d0[0, :, 0:1] + d1[0, :, 0:1]).astype(jnp.float32)
    return lax.rsqrt(jnp.maximum(deg, 1.0))


def _prep_body(x_ref, d0, d1, y_ref, n_ref):
    nrm = _norm_col(d0, d1)
    y_ref[...] = x_ref[...] * nrm
    n_ref[...] = jnp.broadcast_to(nrm, (nrm.shape[0], n_ref.shape[1]))


def _zscale_body(sa, sb, n_ref, z_ref):
    nrm = n_ref[:, 0:1]
    z_ref[...] = (sa[0] + sb[0]) * (nrm * nrm)


def _dot(a, b):
    return jnp.dot(a, b, preferred_element_type=jnp.float32)


def _combine1_body(x_ref, s1a, s1b, s2a, s2b, n_ref,
                   w0, w1, w2, b1, h_ref, y_ref):
    nrm = n_ref[:, 0:1]
    x = x_ref[...]
    p1 = (s1a[0] + s1b[0]) * nrm
    p2 = (s2a[0] + s2b[0]) * nrm
    h = (_dot(x, w0[0]) - _dot(p1, w1[0]) + _dot(2.0 * p2 - x, w2[0])
         + b1[...])
    h_ref[...] = h
    y_ref[...] = h * nrm


def _combine2_body(h_ref, t1a, t1b, t2a, t2b, n_ref,
                   w0, w1, w2, b2, wfc, bfc, o_ref):
    nrm = n_ref[:, 0:1]
    h = h_ref[...]
    p1 = (t1a[0] + t1b[0]) * nrm
    p2 = (t2a[0] + t2b[0]) * nrm
    h2 = (_dot(h, w0[0]) - _dot(p1, w1[0]) + _dot(2.0 * p2 - h, w2[0])
          + b2[...])
    o_ref[...] = _dot(h2, wfc[0]) + bfc[...]


def _row_spec(bn, d):
    return pl.BlockSpec((bn, d), lambda i: (i, 0))


def _half_spec(bn, h, which):
    return pl.BlockSpec((1, bn, h), lambda i, _w=which: (_w, i, 0))


def _w_spec(k):
    return pl.BlockSpec((1, 128, 128), lambda i, _k=k: (_k, 0, 0))


def _bias_spec():
    return pl.BlockSpec((1, 128), lambda i: (0, 0))


def kernel(x, edge_index, W1, b1, W2, b2, Wfc, bfc):
    N, D = x.shape
    E = edge_index.shape[1]
    nch = (E // (NC * NT)) // CH
    eidx = edge_index.reshape(2, NC, NT, NPH, nch // NPH, CH)

    NP = (N + 8 * NT - 1) // (8 * NT) * (8 * NT)
    zerosD = jnp.zeros((NP, D), jnp.float32)
    onesD = jnp.ones((CH, D), jnp.float32)

    degp = _deg_sc(eidx, zerosD, onesD)

    grid = (N // BN,)
    dspecs = [_half_spec(BN, D, 0), _half_spec(BN, D, 1)]
    NW = 8

    y1, nrm = pl.pallas_call(
        _prep_body, grid=grid,
        in_specs=[_row_spec(BN, D)] + dspecs,
        out_specs=[_row_spec(BN, D), _row_spec(BN, NW)],
        out_shape=[jax.ShapeDtypeStruct((N, D), jnp.float32),
                   jax.ShapeDtypeStruct((N, NW), jnp.float32)],
    )(x, degp, degp)

    def seg(table):
        return _segsum_sc(D, table, eidx, zerosD)

    def zscale(sarr):
        return pl.pallas_call(
            _zscale_body, grid=grid,
            in_specs=[_half_spec(BN, D, 0), _half_spec(BN, D, 1),
                      _row_spec(BN, NW)],
            out_specs=_row_spec(BN, D),
            out_shape=jax.ShapeDtypeStruct((N, D), jnp.float32),
        )(sarr, sarr, nrm)

    s1 = seg(y1)
    z1 = zscale(s1)
    s2 = seg(z1)

    h1, y2 = pl.pallas_call(
        _combine1_body, grid=grid,
        in_specs=([_row_spec(BN, D),
                   _half_spec(BN, D, 0), _half_spec(BN, D, 1),
                   _half_spec(BN, D, 0), _half_spec(BN, D, 1),
                   _row_spec(BN, NW)]
                  + [_w_spec(0), _w_spec(1), _w_spec(2), _bias_spec()]),
        out_specs=[_row_spec(BN, D), _row_spec(BN, D)],
        out_shape=[jax.ShapeDtypeStruct((N, D), jnp.float32),
                   jax.ShapeDtypeStruct((N, D), jnp.float32)],
    )(x, s1, s1, s2, s2, nrm, W1, W1, W1, b1.reshape(1, D))

    t1 = seg(y2)
    z2 = zscale(t1)
    t2 = seg(z2)

    out = pl.pallas_call(
        _combine2_body, grid=grid,
        in_specs=([_row_spec(BN, D),
                   _half_spec(BN, D, 0), _half_spec(BN, D, 1),
                   _half_spec(BN, D, 0), _half_spec(BN, D, 1),
                   _row_spec(BN, NW)]
                  + [_w_spec(0), _w_spec(1), _w_spec(2), _bias_spec(),
                     pl.BlockSpec((1, 128, 128), lambda i: (0, 0, 0)),
                     _bias_spec()]),
        out_specs=_row_spec(BN, D),
        out_shape=jax.ShapeDtypeStruct((N, D), jnp.float32),
    )(h1, t1, t1, t2, t2, nrm, W2, W2, W2, b2.reshape(1, D),
      Wfc.reshape(1, D, D), bfc.reshape(1, D))

    return out
